# baseline (speedup 1.0000x reference)
import functools

import jax
import jax.numpy as jnp
from jax import lax
from jax.experimental import pallas as pl
from jax.experimental.pallas import tpu as pltpu
from jax.experimental.pallas import tpu_sc as plsc

N = 10000
NP = 10240
D = 128
E = 320000
C = 40
CHUNK = 128
NW = 32
PW = 80
NCHP = NW * PW
EPAD = NCHP * CHUNK
NB = 4
IB = 16
RB = 10
BR = NP // RB
TPR = NP // 16

_HI = lax.Precision.HIGHEST


def _sc_worker_id():
    cid = lax.axis_index("c")
    sid = lax.axis_index("s")
    return cid, sid, sid * 2 + cid


def _sc_zero_acc(zeros_hbm, acc, sid):
    pltpu.sync_copy(zeros_hbm.at[pl.ds(sid * TPR, TPR)],
                    acc.at[pl.ds(sid * TPR, TPR)])


def _sc_copy_out(acc, out_hbm, cid, sid):
    pltpu.sync_copy(acc.at[pl.ds(sid * TPR, TPR)],
                    out_hbm.at[cid, pl.ds(sid * TPR, TPR)])


def _sc_deg(dst2d, zeros128):
    mesh = plsc.VectorSubcoreMesh(core_axis_name="c", subcore_axis_name="s")

    def body(dst_hbm, zeros_hbm, out_hbm, hist, didx, blk, colbuf, slab):
        cid, sid, gwid = _sc_worker_id()
        base = gwid * PW
        lane = lax.iota(jnp.int32, 16)
        zid16 = lane * 0
        zf16 = zid16.astype(jnp.float32)
        ones16 = zf16 + 1.0

        @pl.loop(0, NP // 16)
        def _(i):
            hist[pl.ds(i * 16, 16)] = zf16

        @pl.loop(0, PW // IB)
        def _(b):
            pltpu.sync_copy(dst_hbm.at[pl.ds(base + b * IB, IB)], didx)
            for r in range(IB):
                for k in range(CHUNK // 16):
                    idx16 = didx[r, pl.ds(k * 16, 16)]
                    plsc.addupdate_scatter(hist, [idx16], ones16)

        pltpu.sync_copy(hist, slab.at[sid])
        plsc.subcore_barrier()

        pltpu.sync_copy(slab.at[:, pl.ds(sid * TPR, TPR)], blk)
        pltpu.sync_copy(zeros_hbm.at[pl.ds(0, TPR)], colbuf)

        for c in range(TPR // 16):
            v = blk[0, pl.ds(c * 16, 16)]
            for r in range(1, 16):
                v = v + blk[r, pl.ds(c * 16, 16)]
            rows = lane + (c * 16)
            plsc.store_scatter(colbuf, [rows, zid16], v)

        pltpu.sync_copy(colbuf, out_hbm.at[cid, pl.ds(sid * TPR, TPR)])

    return pl.kernel(
        body,
        out_type=jax.ShapeDtypeStruct((2, NP, D), jnp.float32),
        mesh=mesh,
        compiler_params=pltpu.CompilerParams(needs_layout_passes=False),
        scratch_types=[
            pltpu.VMEM((NP,), jnp.float32),
            pltpu.VMEM((IB, CHUNK), jnp.int32),
            pltpu.VMEM((16, TPR), jnp.float32),
            pltpu.VMEM((TPR, D), jnp.float32),
            pltpu.VMEM_SHARED((16, NP), jnp.float32),
        ],
    )(dst2d, zeros128)


def _sc_scatter(hp, src2d, dst2d, zeros128):
    mesh = plsc.VectorSubcoreMesh(core_axis_name="c", subcore_axis_name="s")

    def body(hp_hbm, src_hbm, dst_hbm, zeros_hbm, out_hbm,
             acc, sidx, didx, bufs, gsem, ssem):
        cid, sid, gwid = _sc_worker_id()
        base = gwid * PW

        def load_idx(blk):
            s = lax.rem(blk, 2)
            pltpu.sync_copy(src_hbm.at[pl.ds(base + blk * IB, IB)], sidx.at[s])
            pltpu.sync_copy(dst_hbm.at[pl.ds(base + blk * IB, IB)], didx.at[s])

        def gath(j, p):
            s, r = lax.rem(j // IB, 2), lax.rem(j, IB)
            for h in range(2):
                idx = sidx.at[s, r, pl.ds(h * 64, 64)]
                pltpu.async_copy(hp_hbm.at[idx],
                                 bufs.at[p, pl.ds(h * 64, 64)], gsem)

        def scat(j, p):
            idx = didx.at[lax.rem(j // IB, 2), lax.rem(j, IB)]
            pltpu.async_copy(bufs.at[p], acc.at[idx], ssem, add=True)

        def gwait():
            for h in range(2):
                pltpu.make_async_copy(
                    hp_hbm.at[sidx.at[0, 0, pl.ds(0, 64)]],
                    bufs.at[0, pl.ds(0, 64)], gsem).wait()

        def swait():
            pltpu.make_async_copy(bufs.at[0], acc.at[didx.at[0, 0]],
                                  ssem).wait()

        load_idx(0)
        _sc_zero_acc(zeros_hbm, acc, sid)
        plsc.subcore_barrier()
        gath(0, 0)

        @pl.loop(0, PW)
        def _(j):
            p = lax.rem(j, 2)

            @pl.when(j > 0)
            def _():
                swait()

            nxt = j + 1

            @pl.when(nxt < PW)
            def _():
                @pl.when(lax.rem(nxt, IB) == 0)
                def _():
                    load_idx(nxt // IB)

                gath(nxt, 1 - p)

            gwait()
            scat(j, p)

        swait()
        plsc.subcore_barrier()
        _sc_copy_out(acc, out_hbm, cid, sid)

    return pl.kernel(
        body,
        out_type=jax.ShapeDtypeStruct((2, NP, D), jnp.float32),
        mesh=mesh,
        scratch_types=[
            pltpu.VMEM_SHARED((NP, D), jnp.float32),
            pltpu.VMEM((2, IB, CHUNK), jnp.int32),
            pltpu.VMEM((2, IB, CHUNK), jnp.int32),
            pltpu.VMEM((2, CHUNK, D), jnp.float32),
            pltpu.SemaphoreType.DMA,
            pltpu.SemaphoreType.DMA,
        ],
    )(hp, src2d, dst2d, zeros128)


def _dis_body(deg_ref, o_ref):
    o_ref[...] = lax.rsqrt(1.0 + deg_ref[0, :, 0:1] + deg_ref[1, :, 0:1])


def _mm_t(a, w):
    return lax.dot_general(a, w, (((1,), (1,)), ((), ())),
                           precision=_HI, preferred_element_type=jnp.float32)


def _fuse_a_body(x_ref, wi_ref, bi_ref, w0_ref, dis_ref, o_ref):
    x0 = _mm_t(x_ref[...], wi_ref[...]) + bi_ref[...]
    o_ref[...] = _mm_t(x0, w0_ref[...]) * dis_ref[...]


def _fuse_b_body(s_ref, hp_ref, dis_ref, b_ref, w_ref, o_ref):
    dis = dis_ref[...]
    x = (s_ref[0] + s_ref[1] + hp_ref[...]) * dis + b_ref[...]
    x = jnp.maximum(x, 0.0)
    o_ref[...] = _mm_t(x, w_ref[...]) * dis


def _fuse_d_body(s_ref, hp_ref, dis_ref, b_ref, wl_ref, bl_ref,
                 emb_ref, logp_ref):
    x = (s_ref[0] + s_ref[1] + hp_ref[...]) * dis_ref[...] + b_ref[...]
    e = _mm_t(x, wl_ref[...]) + bl_ref[...]
    emb_ref[...] = e
    m = jnp.max(e, axis=1, keepdims=True)
    s = e - m
    logp_ref[...] = s - jnp.log(jnp.sum(jnp.exp(s), axis=1, keepdims=True))


_ROWS = pl.BlockSpec((BR, D), lambda i: (i, 0))
_WMAT = pl.BlockSpec((D, D), lambda i: (0, 0))
_BIAS = pl.BlockSpec((1, D), lambda i: (0, 0))
_DEG2 = pl.BlockSpec((2, BR, D), lambda i: (0, i, 0))
_S2 = pl.BlockSpec((2, BR, D), lambda i: (0, i, 0))
_DIS = pl.BlockSpec((BR, 1), lambda i: (i, 0))
BRF = N // RB


def _tc_dis(deg2):
    return pl.pallas_call(
        _dis_body, grid=(RB,),
        in_specs=[_DEG2], out_specs=_DIS,
        out_shape=jax.ShapeDtypeStruct((NP, 1), jnp.float32),
    )(deg2)


def _tc_fuse_a(x, wi, bi, w0, dis):
    return pl.pallas_call(
        _fuse_a_body, grid=(RB,),
        in_specs=[_ROWS, _WMAT, _BIAS, _WMAT, _DIS], out_specs=_ROWS,
        out_shape=jax.ShapeDtypeStruct((NP, D), jnp.float32),
    )(x, wi, bi, w0, dis)


def _tc_fuse_b(s2, hp, dis, b, w):
    return pl.pallas_call(
        _fuse_b_body, grid=(RB,),
        in_specs=[_S2, _ROWS, _DIS, _BIAS, _WMAT], out_specs=_ROWS,
        out_shape=jax.ShapeDtypeStruct((NP, D), jnp.float32),
    )(s2, hp, dis, b, w)


def _tc_fuse_d(s2, hp, dis, b, wl, bl):
    outspec = pl.BlockSpec((BRF, C), lambda i: (i, 0))
    return pl.pallas_call(
        _fuse_d_body, grid=(RB,),
        in_specs=[pl.BlockSpec((2, BRF, D), lambda i: (0, i, 0)),
                  pl.BlockSpec((BRF, D), lambda i: (i, 0)),
                  pl.BlockSpec((BRF, 1), lambda i: (i, 0)),
                  _BIAS,
                  pl.BlockSpec((C, D), lambda i: (0, 0)),
                  pl.BlockSpec((1, C), lambda i: (0, 0))],
        out_specs=[outspec, outspec],
        out_shape=[jax.ShapeDtypeStruct((N, C), jnp.float32),
                   jax.ShapeDtypeStruct((N, C), jnp.float32)],
    )(s2, hp, dis, b, wl, bl)


def kernel(x_h, adj, edge_index, pos_feat, W_init, b_init,
           Wg0, bg0, Wg1, bg1, Wg2, bg2, W_last, b_last):
    fake = N + (jnp.arange(EPAD - E, dtype=jnp.int32) % (NP - N))
    src2d = jnp.concatenate([edge_index[0], fake]).reshape(NCHP, CHUNK)
    dst2d = jnp.concatenate([edge_index[1], fake]).reshape(NCHP, CHUNK)
    xp = jnp.pad(x_h, ((0, NP - N), (0, 0)))
    zeros128 = jnp.zeros((NP, D), jnp.float32)

    deg2 = _sc_deg(dst2d, zeros128)
    dis = _tc_dis(deg2)
    hp = _tc_fuse_a(xp, W_init, b_init.reshape(1, D), Wg0, dis)
    for i, b, w_next in ((0, bg0, Wg1), (1, bg1, Wg2)):
        s2 = _sc_scatter(hp, src2d, dst2d, zeros128)
        hp = _tc_fuse_b(s2, hp, dis, b.reshape(1, D), w_next)
    s2 = _sc_scatter(hp, src2d, dst2d, zeros128)
    emb, logp = _tc_fuse_d(s2, hp, dis, bg2.reshape(1, D),
                           W_last, b_last.reshape(1, C))
    return (emb, logp)


# speedup vs baseline: 1.0194x; 1.0194x over previous
import functools

import jax
import jax.numpy as jnp
from jax import lax
from jax.experimental import pallas as pl
from jax.experimental.pallas import tpu as pltpu
from jax.experimental.pallas import tpu_sc as plsc

N = 10000
NP = 10240
D = 128
E = 320000
C = 40
CHUNK = 128
NW = 32
PW = 80
NCHP = NW * PW
EPAD = NCHP * CHUNK
NB = 4
IB = 16
RB = 10
BR = NP // RB
TPR = NP // 16

_HI = lax.Precision.HIGHEST


def _sc_worker_id():
    cid = lax.axis_index("c")
    sid = lax.axis_index("s")
    return cid, sid, sid * 2 + cid


def _sc_zero_acc(zeros_hbm, acc, sid):
    pltpu.sync_copy(zeros_hbm.at[pl.ds(sid * TPR, TPR)],
                    acc.at[pl.ds(sid * TPR, TPR)])


def _sc_copy_out(acc, out_hbm, cid, sid):
    pltpu.sync_copy(acc.at[pl.ds(sid * TPR, TPR)],
                    out_hbm.at[cid, pl.ds(sid * TPR, TPR)])


def _sc_deg(dst2d, zeros128):
    mesh = plsc.VectorSubcoreMesh(core_axis_name="c", subcore_axis_name="s")

    def body(dst_hbm, zeros_hbm, out_hbm, hist, didx, blk, colbuf, slab):
        cid, sid, gwid = _sc_worker_id()
        base = gwid * PW
        lane = lax.iota(jnp.int32, 16)
        zid16 = lane * 0
        zf16 = zid16.astype(jnp.float32)
        ones16 = zf16 + 1.0

        @pl.loop(0, NP // 16)
        def _(i):
            hist[pl.ds(i * 16, 16)] = zf16

        @pl.loop(0, PW // IB)
        def _(b):
            pltpu.sync_copy(dst_hbm.at[pl.ds(base + b * IB, IB)], didx)
            for r in range(IB):
                for k in range(CHUNK // 16):
                    idx16 = didx[r, pl.ds(k * 16, 16)]
                    plsc.addupdate_scatter(hist, [idx16], ones16)

        pltpu.sync_copy(hist, slab.at[sid])
        plsc.subcore_barrier()

        pltpu.sync_copy(slab.at[:, pl.ds(sid * TPR, TPR)], blk)
        pltpu.sync_copy(zeros_hbm.at[pl.ds(0, TPR)], colbuf)

        for c in range(TPR // 16):
            v = blk[0, pl.ds(c * 16, 16)]
            for r in range(1, 16):
                v = v + blk[r, pl.ds(c * 16, 16)]
            rows = lane + (c * 16)
            plsc.store_scatter(colbuf, [rows, zid16], v)

        pltpu.sync_copy(colbuf, out_hbm.at[cid, pl.ds(sid * TPR, TPR)])

    return pl.kernel(
        body,
        out_type=jax.ShapeDtypeStruct((2, NP, D), jnp.float32),
        mesh=mesh,
        compiler_params=pltpu.CompilerParams(needs_layout_passes=False),
        scratch_types=[
            pltpu.VMEM((NP,), jnp.float32),
            pltpu.VMEM((IB, CHUNK), jnp.int32),
            pltpu.VMEM((16, TPR), jnp.float32),
            pltpu.VMEM((TPR, D), jnp.float32),
            pltpu.VMEM_SHARED((16, NP), jnp.float32),
        ],
    )(dst2d, zeros128)


def _sc_scatter(hp, src2d, dst2d, zeros128):
    mesh = plsc.VectorSubcoreMesh(core_axis_name="c", subcore_axis_name="s")

    def body(hp_hbm, src_hbm, dst_hbm, zeros_hbm, out_hbm,
             acc, sidx, didx, bufs, gsem, ssem):
        cid, sid, gwid = _sc_worker_id()
        base = gwid * PW

        def load_idx(blk):
            s = lax.rem(blk, 2)
            pltpu.sync_copy(src_hbm.at[pl.ds(base + blk * IB, IB)], sidx.at[s])
            pltpu.sync_copy(dst_hbm.at[pl.ds(base + blk * IB, IB)], didx.at[s])

        def gath(j, p):
            s, r = lax.rem(j // IB, 2), lax.rem(j, IB)
            for h in range(2):
                idx = sidx.at[s, r, pl.ds(h * 64, 64)]
                pltpu.async_copy(hp_hbm.at[idx],
                                 bufs.at[p, pl.ds(h * 64, 64)], gsem)

        def scat(j, p):
            idx = didx.at[lax.rem(j // IB, 2), lax.rem(j, IB)]
            pltpu.async_copy(bufs.at[p], acc.at[idx], ssem, add=True)

        def gwait():
            for h in range(2):
                pltpu.make_async_copy(
                    hp_hbm.at[sidx.at[0, 0, pl.ds(0, 64)]],
                    bufs.at[0, pl.ds(0, 64)], gsem).wait()

        def swait():
            pltpu.make_async_copy(bufs.at[0], acc.at[didx.at[0, 0]],
                                  ssem).wait()

        load_idx(0)
        _sc_zero_acc(zeros_hbm, acc, sid)
        plsc.subcore_barrier()
        gath(0, 0)

        @pl.loop(0, PW)
        def _(j):
            p = lax.rem(j, 2)

            @pl.when(j > 0)
            def _():
                swait()

            nxt = j + 1

            @pl.when(nxt < PW)
            def _():
                @pl.when(lax.rem(nxt, IB) == 0)
                def _():
                    load_idx(nxt // IB)

                gath(nxt, 1 - p)

            gwait()
            scat(j, p)

        swait()
        plsc.subcore_barrier()
        _sc_copy_out(acc, out_hbm, cid, sid)

    return pl.kernel(
        body,
        out_type=jax.ShapeDtypeStruct((2, NP, D), jnp.float32),
        mesh=mesh,
        scratch_types=[
            pltpu.VMEM_SHARED((NP, D), jnp.float32),
            pltpu.VMEM((2, IB, CHUNK), jnp.int32),
            pltpu.VMEM((2, IB, CHUNK), jnp.int32),
            pltpu.VMEM((2, CHUNK, D), jnp.float32),
            pltpu.SemaphoreType.DMA,
            pltpu.SemaphoreType.DMA,
        ],
    )(hp, src2d, dst2d, zeros128)


def _mm_t(a, w):
    return lax.dot_general(a, w, (((1,), (1,)), ((), ())),
                           precision=_HI, preferred_element_type=jnp.float32)


def _fuse_a_body(x_ref, wi_ref, bi_ref, w0_ref, deg_ref, o_ref, dis_ref):
    dis = lax.rsqrt(1.0 + deg_ref[0, :, 0:1] + deg_ref[1, :, 0:1])
    dis_ref[...] = dis
    x0 = _mm_t(x_ref[...], wi_ref[...]) + bi_ref[...]
    o_ref[...] = _mm_t(x0, w0_ref[...]) * dis


def _fuse_b_body(s_ref, hp_ref, dis_ref, b_ref, w_ref, o_ref):
    dis = dis_ref[...]
    x = (s_ref[0] + s_ref[1] + hp_ref[...]) * dis + b_ref[...]
    x = jnp.maximum(x, 0.0)
    o_ref[...] = _mm_t(x, w_ref[...]) * dis


def _fuse_d_body(s_ref, hp_ref, dis_ref, b_ref, wl_ref, bl_ref,
                 emb_ref, logp_ref):
    x = (s_ref[0] + s_ref[1] + hp_ref[...]) * dis_ref[...] + b_ref[...]
    e = _mm_t(x, wl_ref[...]) + bl_ref[...]
    emb_ref[...] = e
    m = jnp.max(e, axis=1, keepdims=True)
    s = e - m
    logp_ref[...] = s - jnp.log(jnp.sum(jnp.exp(s), axis=1, keepdims=True))


_ROWS = pl.BlockSpec((BR, D), lambda i: (i, 0))
_WMAT = pl.BlockSpec((D, D), lambda i: (0, 0))
_BIAS = pl.BlockSpec((1, D), lambda i: (0, 0))
_DEG2 = pl.BlockSpec((2, BR, D), lambda i: (0, i, 0))
_S2 = pl.BlockSpec((2, BR, D), lambda i: (0, i, 0))
_DIS = pl.BlockSpec((BR, 1), lambda i: (i, 0))
BRF = N // RB


def _tc_fuse_a(x, wi, bi, w0, deg2):
    return pl.pallas_call(
        _fuse_a_body, grid=(RB,),
        in_specs=[_ROWS, _WMAT, _BIAS, _WMAT, _DEG2],
        out_specs=[_ROWS, _DIS],
        out_shape=[jax.ShapeDtypeStruct((NP, D), jnp.float32),
                   jax.ShapeDtypeStruct((NP, 1), jnp.float32)],
    )(x, wi, bi, w0, deg2)


def _tc_fuse_b(s2, hp, dis, b, w):
    return pl.pallas_call(
        _fuse_b_body, grid=(RB,),
        in_specs=[_S2, _ROWS, _DIS, _BIAS, _WMAT], out_specs=_ROWS,
        out_shape=jax.ShapeDtypeStruct((NP, D), jnp.float32),
    )(s2, hp, dis, b, w)


def _tc_fuse_d(s2, hp, dis, b, wl, bl):
    outspec = pl.BlockSpec((BRF, C), lambda i: (i, 0))
    return pl.pallas_call(
        _fuse_d_body, grid=(RB,),
        in_specs=[pl.BlockSpec((2, BRF, D), lambda i: (0, i, 0)),
                  pl.BlockSpec((BRF, D), lambda i: (i, 0)),
                  pl.BlockSpec((BRF, 1), lambda i: (i, 0)),
                  _BIAS,
                  pl.BlockSpec((C, D), lambda i: (0, 0)),
                  pl.BlockSpec((1, C), lambda i: (0, 0))],
        out_specs=[outspec, outspec],
        out_shape=[jax.ShapeDtypeStruct((N, C), jnp.float32),
                   jax.ShapeDtypeStruct((N, C), jnp.float32)],
    )(s2, hp, dis, b, wl, bl)


def kernel(x_h, adj, edge_index, pos_feat, W_init, b_init,
           Wg0, bg0, Wg1, bg1, Wg2, bg2, W_last, b_last):
    fake = N + (jnp.arange(EPAD - E, dtype=jnp.int32) % (NP - N))
    src2d = jnp.concatenate([edge_index[0], fake]).reshape(NCHP, CHUNK)
    dst2d = jnp.concatenate([edge_index[1], fake]).reshape(NCHP, CHUNK)
    xp = jnp.pad(x_h, ((0, NP - N), (0, 0)))
    zeros128 = jnp.zeros((NP, D), jnp.float32)

    deg2 = _sc_deg(dst2d, zeros128)
    hp, dis = _tc_fuse_a(xp, W_init, b_init.reshape(1, D), Wg0, deg2)
    for i, b, w_next in ((0, bg0, Wg1), (1, bg1, Wg2)):
        s2 = _sc_scatter(hp, src2d, dst2d, zeros128)
        hp = _tc_fuse_b(s2, hp, dis, b.reshape(1, D), w_next)
    s2 = _sc_scatter(hp, src2d, dst2d, zeros128)
    emb, logp = _tc_fuse_d(s2, hp, dis, bg2.reshape(1, D),
                           W_last, b_last.reshape(1, C))
    return (emb, logp)


# speedup vs baseline: 1.0198x; 1.0003x over previous
import jax
import jax.numpy as jnp
from jax import lax
from jax.experimental import pallas as pl
from jax.experimental.pallas import tpu as pltpu
from jax.experimental.pallas import tpu_sc as plsc

N = 10000
NP = 10240
D = 128
E = 320000
C = 40
CHUNK = 128
NW = 32
PW = 80
NCHP = NW * PW
EPAD = NCHP * CHUNK
IB = 16
RB = 10
BR = NP // RB
TPR = NP // 16

_HI = lax.Precision.HIGHEST


def _sc_worker_id():
    cid = lax.axis_index("c")
    sid = lax.axis_index("s")
    return cid, sid, sid * 2 + cid


def _sc_zero_acc(zeros_hbm, acc, sid):
    pltpu.sync_copy(zeros_hbm.at[pl.ds(sid * TPR, TPR)],
                    acc.at[pl.ds(sid * TPR, TPR)])


def _sc_copy_out(acc, out_hbm, cid, sid):
    pltpu.sync_copy(acc.at[pl.ds(sid * TPR, TPR)],
                    out_hbm.at[cid, pl.ds(sid * TPR, TPR)])


def _sc_deg(dst2d, zeros128):
    mesh = plsc.VectorSubcoreMesh(core_axis_name="c", subcore_axis_name="s")

    def body(dst_hbm, zeros_hbm, out_hbm, hist, didx, blk, colbuf, slab):
        cid, sid, gwid = _sc_worker_id()
        base = gwid * PW
        lane = lax.iota(jnp.int32, 16)
        zid16 = lane * 0
        zf16 = zid16.astype(jnp.float32)
        ones16 = zf16 + 1.0

        @pl.loop(0, NP // 16)
        def _(i):
            hist[pl.ds(i * 16, 16)] = zf16

        @pl.loop(0, PW // IB)
        def _(b):
            pltpu.sync_copy(dst_hbm.at[pl.ds(base + b * IB, IB)], didx)
            for r in range(IB):
                for k in range(CHUNK // 16):
                    idx16 = didx[r, pl.ds(k * 16, 16)]
                    plsc.addupdate_scatter(hist, [idx16], ones16)

        pltpu.sync_copy(hist, slab.at[sid])
        plsc.subcore_barrier()

        pltpu.sync_copy(slab.at[:, pl.ds(sid * TPR, TPR)], blk)
        pltpu.sync_copy(zeros_hbm.at[pl.ds(0, TPR)], colbuf)

        for c in range(TPR // 16):
            v = blk[0, pl.ds(c * 16, 16)]
            for r in range(1, 16):
                v = v + blk[r, pl.ds(c * 16, 16)]
            rows = lane + (c * 16)
            plsc.store_scatter(colbuf, [rows, zid16], v)

        pltpu.sync_copy(colbuf, out_hbm.at[cid, pl.ds(sid * TPR, TPR)])

    return pl.kernel(
        body,
        out_type=jax.ShapeDtypeStruct((2, NP, D), jnp.float32),
        mesh=mesh,
        compiler_params=pltpu.CompilerParams(needs_layout_passes=False),
        scratch_types=[
            pltpu.VMEM((NP,), jnp.float32),
            pltpu.VMEM((IB, CHUNK), jnp.int32),
            pltpu.VMEM((16, TPR), jnp.float32),
            pltpu.VMEM((TPR, D), jnp.float32),
            pltpu.VMEM_SHARED((16, NP), jnp.float32),
        ],
    )(dst2d, zeros128)


def _sc_scatter(hp, src2d, dst2d, zeros128):
    mesh = plsc.VectorSubcoreMesh(core_axis_name="c", subcore_axis_name="s")

    def body(hp_hbm, src_hbm, dst_hbm, zeros_hbm, out_hbm,
             acc, sidx, didx, bufs, gsem, ssem):
        cid, sid, gwid = _sc_worker_id()
        base = gwid * PW

        def load_idx(blk):
            s = lax.rem(blk, 2)
            pltpu.sync_copy(src_hbm.at[pl.ds(base + blk * IB, IB)], sidx.at[s])
            pltpu.sync_copy(dst_hbm.at[pl.ds(base + blk * IB, IB)], didx.at[s])

        def gath(j, p):
            s, r = lax.rem(j // IB, 2), lax.rem(j, IB)
            for h in range(2):
                idx = sidx.at[s, r, pl.ds(h * 64, 64)]
                pltpu.async_copy(hp_hbm.at[idx],
                                 bufs.at[p, pl.ds(h * 64, 64)], gsem)

        def scat(j, p):
            idx = didx.at[lax.rem(j // IB, 2), lax.rem(j, IB)]
            pltpu.async_copy(bufs.at[p], acc.at[idx], ssem, add=True)

        def gwait():
            for h in range(2):
                pltpu.make_async_copy(
                    hp_hbm.at[sidx.at[0, 0, pl.ds(0, 64)]],
                    bufs.at[0, pl.ds(0, 64)], gsem).wait()

        def swait():
            pltpu.make_async_copy(bufs.at[0], acc.at[didx.at[0, 0]],
                                  ssem).wait()

        load_idx(0)
        _sc_zero_acc(zeros_hbm, acc, sid)
        plsc.subcore_barrier()
        gath(0, 0)

        @pl.loop(0, PW)
        def _(j):
            p = lax.rem(j, 2)

            @pl.when(j > 0)
            def _():
                swait()

            nxt = j + 1

            @pl.when(nxt < PW)
            def _():
                @pl.when(lax.rem(nxt, IB) == 0)
                def _():
                    load_idx(nxt // IB)

                gath(nxt, 1 - p)

            gwait()
            scat(j, p)

        swait()
        plsc.subcore_barrier()
        _sc_copy_out(acc, out_hbm, cid, sid)

    return pl.kernel(
        body,
        out_type=jax.ShapeDtypeStruct((2, NP, D), jnp.float32),
        mesh=mesh,
        scratch_types=[
            pltpu.VMEM_SHARED((NP, D), jnp.float32),
            pltpu.VMEM((2, IB, CHUNK), jnp.int32),
            pltpu.VMEM((2, IB, CHUNK), jnp.int32),
            pltpu.VMEM((2, CHUNK, D), jnp.float32),
            pltpu.SemaphoreType.DMA,
            pltpu.SemaphoreType.DMA,
        ],
    )(hp, src2d, dst2d, zeros128)


def _mm_t(a, w):
    return lax.dot_general(a, w, (((1,), (1,)), ((), ())),
                           precision=_HI, preferred_element_type=jnp.float32)


def _fuse_a_body(x_ref, wi_ref, bi_ref, w0_ref, deg_ref, o_ref, dis_ref):
    dis = lax.rsqrt(1.0 + deg_ref[0, :, 0:1] + deg_ref[1, :, 0:1])
    dis_ref[...] = dis
    x0 = _mm_t(x_ref[...], wi_ref[...]) + bi_ref[...]
    o_ref[...] = _mm_t(x0, w0_ref[...]) * dis


def _fuse_b_body(s_ref, hp_ref, dis_ref, b_ref, w_ref, o_ref):
    dis = dis_ref[...]
    x = (s_ref[0] + s_ref[1] + hp_ref[...]) * dis + b_ref[...]
    x = jnp.maximum(x, 0.0)
    o_ref[...] = _mm_t(x, w_ref[...]) * dis


def _fuse_d_body(s_ref, hp_ref, dis_ref, b_ref, wl_ref, bl_ref,
                 emb_ref, logp_ref):
    x = (s_ref[0] + s_ref[1] + hp_ref[...]) * dis_ref[...] + b_ref[...]
    e = _mm_t(x, wl_ref[...]) + bl_ref[...]
    emb_ref[...] = e
    m = jnp.max(e, axis=1, keepdims=True)
    s = e - m
    logp_ref[...] = s - jnp.log(jnp.sum(jnp.exp(s), axis=1, keepdims=True))


_ROWS = pl.BlockSpec((BR, D), lambda i: (i, 0))
_WMAT = pl.BlockSpec((D, D), lambda i: (0, 0))
_BIAS = pl.BlockSpec((1, D), lambda i: (0, 0))
_DEG2 = pl.BlockSpec((2, BR, D), lambda i: (0, i, 0))
_S2 = pl.BlockSpec((2, BR, D), lambda i: (0, i, 0))
_DIS = pl.BlockSpec((BR, 1), lambda i: (i, 0))
BRF = N // RB


def _tc_fuse_a(x, wi, bi, w0, deg2):
    return pl.pallas_call(
        _fuse_a_body, grid=(RB,),
        in_specs=[_ROWS, _WMAT, _BIAS, _WMAT, _DEG2],
        out_specs=[_ROWS, _DIS],
        out_shape=[jax.ShapeDtypeStruct((NP, D), jnp.float32),
                   jax.ShapeDtypeStruct((NP, 1), jnp.float32)],
    )(x, wi, bi, w0, deg2)


def _tc_fuse_b(s2, hp, dis, b, w):
    return pl.pallas_call(
        _fuse_b_body, grid=(RB,),
        in_specs=[_S2, _ROWS, _DIS, _BIAS, _WMAT], out_specs=_ROWS,
        out_shape=jax.ShapeDtypeStruct((NP, D), jnp.float32),
    )(s2, hp, dis, b, w)


def _tc_fuse_d(s2, hp, dis, b, wl, bl):
    outspec = pl.BlockSpec((BRF, C), lambda i: (i, 0))
    return pl.pallas_call(
        _fuse_d_body, grid=(RB,),
        in_specs=[pl.BlockSpec((2, BRF, D), lambda i: (0, i, 0)),
                  pl.BlockSpec((BRF, D), lambda i: (i, 0)),
                  pl.BlockSpec((BRF, 1), lambda i: (i, 0)),
                  _BIAS,
                  pl.BlockSpec((C, D), lambda i: (0, 0)),
                  pl.BlockSpec((1, C), lambda i: (0, 0))],
        out_specs=[outspec, outspec],
        out_shape=[jax.ShapeDtypeStruct((N, C), jnp.float32),
                   jax.ShapeDtypeStruct((N, C), jnp.float32)],
    )(s2, hp, dis, b, wl, bl)


def kernel(x_h, adj, edge_index, pos_feat, W_init, b_init,
           Wg0, bg0, Wg1, bg1, Wg2, bg2, W_last, b_last):
    fake = N + (jnp.arange(EPAD - E, dtype=jnp.int32) % (NP - N))
    src2d = jnp.concatenate([edge_index[0], fake]).reshape(NCHP, CHUNK)
    dst2d = jnp.concatenate([edge_index[1], fake]).reshape(NCHP, CHUNK)
    xp = jnp.pad(x_h, ((0, NP - N), (0, 0)))
    zeros128 = jnp.zeros((NP, D), jnp.float32)

    deg2 = _sc_deg(dst2d, zeros128)
    hp, dis = _tc_fuse_a(xp, W_init, b_init.reshape(1, D), Wg0, deg2)
    for i, b, w_next in ((0, bg0, Wg1), (1, bg1, Wg2)):
        s2 = _sc_scatter(hp, src2d, dst2d, zeros128)
        hp = _tc_fuse_b(s2, hp, dis, b.reshape(1, D), w_next)
    s2 = _sc_scatter(hp, src2d, dst2d, zeros128)
    emb, logp = _tc_fuse_d(s2, hp, dis, bg2.reshape(1, D),
                           W_last, b_last.reshape(1, C))
    return (emb, logp)
